# baseline (speedup 1.0000x reference)
import functools

import jax
import jax.numpy as jnp
from jax import lax
from jax.experimental import pallas as pl
from jax.experimental.pallas import tpu as pltpu
from jax.experimental.pallas import tpu_sc as plsc

N_BITS_K = 1024
HASH_BITS_K = 14
N_HASH_K = 3
BATCH_K = 16384
RAM_K = 2 ** HASH_BITS_K
NW_K = 1024 // 4


_BB = 512


def _addr_body(pos_ref, bits_ref, o0, o1, o2):
    b = bits_ref[...]
    lane = lax.broadcasted_iota(jnp.int32, (8, N_BITS_K), 1)
    outs = (o0, o1, o2)
    for h in range(N_HASH_K):
        w = jnp.zeros((8, N_BITS_K), jnp.int32)
        for j in range(HASH_BITS_K):
            c = N_BITS_K - 1 - pos_ref[h, j]
            w = w + jnp.where(lane == c, 1 << (HASH_BITS_K - 1 - j), 0)
        acc = jnp.sum(b * w[0:1, :], axis=1)
        outs[h][...] = acc + h * RAM_K


def _addr_call(positions, bits):
    grid = (BATCH_K // _BB,)
    out = jax.ShapeDtypeStruct((BATCH_K,), jnp.int32)
    return pl.pallas_call(
        _addr_body,
        grid=grid,
        in_specs=[
            pl.BlockSpec(memory_space=pltpu.SMEM),
            pl.BlockSpec((_BB, N_BITS_K), lambda i: (i, 0)),
        ],
        out_specs=[
            pl.BlockSpec((_BB,), lambda i: (i,)),
            pl.BlockSpec((_BB,), lambda i: (i,)),
            pl.BlockSpec((_BB,), lambda i: (i,)),
        ],
        out_shape=[out, out, out],
    )(positions, bits)


_BN = 512
_BA = 256


def _tr_body(s0_ref, s1_ref, s2_ref, s3_ref, mem_ref, out_ref):
    x = mem_ref[0]
    srefs = (s0_ref, s1_ref, s2_ref, s3_ref)
    acc = None
    for r in range(4):
        z = jnp.dot(srefs[r][...], x, preferred_element_type=jnp.float32)
        t = z.T.astype(jnp.int32) << (8 * r)
        acc = t if acc is None else acc + t
    out_ref[0] = acc


def _pack_mats():
    k = jnp.arange(_BN // 4)[:, None]
    n = jnp.arange(_BN)[None, :]
    return [(n == 4 * k + r).astype(jnp.float32) for r in range(4)]


def _tr_call(memory):
    grid = (N_HASH_K, N_BITS_K // _BN, RAM_K // _BA)
    smats = _pack_mats()
    sspec = pl.BlockSpec((_BN // 4, _BN), lambda h, nb, ab: (0, 0))
    return pl.pallas_call(
        _tr_body,
        grid=grid,
        in_specs=[sspec, sspec, sspec, sspec,
                  pl.BlockSpec((1, _BN, _BA), lambda h, nb, ab: (h, nb, ab))],
        out_specs=pl.BlockSpec((1, _BA, _BN // 4), lambda h, nb, ab: (h, ab, nb)),
        out_shape=jax.ShapeDtypeStruct((N_HASH_K, RAM_K, NW_K), jnp.int32),
    )(*smats, memory)


_NSC = 32
_BPW = BATCH_K // _NSC
_CH = 128
_NCH = _BPW // _CH
_SG = 8
_SR = _CH // _SG


def _sc_body(table_hbm, a0_hbm, a1_hbm, a2_hbm, out_hbm, i0, i1, i2, b0, b1, b2, sem):
    wid = lax.axis_index("s") * 2 + lax.axis_index("c")
    base = wid * _BPW
    pltpu.sync_copy(a0_hbm.at[pl.ds(base, _BPW)], i0)
    pltpu.sync_copy(a1_hbm.at[pl.ds(base, _BPW)], i1)
    pltpu.sync_copy(a2_hbm.at[pl.ds(base, _BPW)], i2)

    for c in range(_NCH):
        copies = []
        for s in range(_SG):
            sl = pl.ds(c * _CH + s * _SR, _SR)
            copies.append(pltpu.async_copy(table_hbm.at[i0.at[sl]], b0.at[s], sem))
            copies.append(pltpu.async_copy(table_hbm.at[i1.at[sl]], b1.at[s], sem))
            copies.append(pltpu.async_copy(table_hbm.at[i2.at[sl]], b2.at[s], sem))
        for cp in copies:
            cp.wait()

        def maj_body(s, _):
            for r in range(_SR):
                for j in range(NW_K // 16):
                    sl = pl.ds(j * 16, 16)
                    v0 = b0[s, r, sl]
                    v1 = b1[s, r, sl]
                    v2 = b2[s, r, sl]
                    b0[s, r, sl] = (v0 & v1) | (v2 & (v0 | v1))
            return 0

        lax.fori_loop(0, _SG, maj_body, 0)
        pltpu.sync_copy(
            b0, out_hbm.at[pl.ds((base + c * _CH) // _SR, _SG)])


_sc_call = functools.partial(
    pl.kernel,
    out_type=jax.ShapeDtypeStruct((BATCH_K // _SR, _SR, NW_K), jnp.int32),
    mesh=plsc.VectorSubcoreMesh(core_axis_name="c", subcore_axis_name="s"),
    scratch_types=[
        pltpu.VMEM((_BPW,), jnp.int32),
        pltpu.VMEM((_BPW,), jnp.int32),
        pltpu.VMEM((_BPW,), jnp.int32),
        pltpu.VMEM((_SG, _SR, NW_K), jnp.int32),
        pltpu.VMEM((_SG, _SR, NW_K), jnp.int32),
        pltpu.VMEM((_SG, _SR, NW_K), jnp.int32),
        pltpu.SemaphoreType.DMA,
    ],
)(_sc_body)


def kernel(bits, memory, positions):
    a0, a1, a2 = _addr_call(positions, bits)
    table = _tr_call(memory).reshape(N_HASH_K * RAM_K, NW_K)
    packed = _sc_call(table, a0, a1, a2)
    out32 = packed.reshape(BATCH_K, NW_K)
    return lax.bitcast_convert_type(out32, jnp.uint8).reshape(BATCH_K, N_BITS_K)
